# baseline (speedup 1.0000x reference)
import functools

import jax
import jax.numpy as jnp
from jax import lax
from jax.experimental import pallas as pl
from jax.experimental.pallas import tpu as pltpu
from jax.experimental.pallas import tpu_sc as plsc

VOCAB = 1000000
EMBED_DIM = 32
BATCH = 16384
ROW_W = 128
PER_ROW = ROW_W // EMBED_DIM
CHUNK = 256
LANES = 16


def _make_gather():
    info = plsc.get_sparse_core_info()
    nc, ns = info.num_cores, info.num_subcores
    nw = nc * ns
    b_per_w = BATCH // nw

    mesh = plsc.VectorSubcoreMesh(core_axis_name="c", subcore_axis_name="s")

    @functools.partial(
        pl.kernel,
        mesh=mesh,
        out_type=jax.ShapeDtypeStruct((BATCH, EMBED_DIM), jnp.float32),
        scratch_types=[
            pltpu.VMEM((b_per_w,), jnp.int32),
            pltpu.VMEM((b_per_w,), jnp.int32),
            pltpu.VMEM((CHUNK, ROW_W), jnp.float32),
            pltpu.VMEM((b_per_w, EMBED_DIM), jnp.float32),
            pltpu.SemaphoreType.DMA,
        ],
        compiler_params=pltpu.CompilerParams(needs_layout_passes=False),
    )
    def gather_kernel(q_hbm, off_hbm, table_hbm, out_hbm,
                      q_v, off_v, rows_v, out_v, sem):
        wid = lax.axis_index("s") * nc + lax.axis_index("c")
        base = wid * b_per_w
        pltpu.sync_copy(q_hbm.at[pl.ds(base, b_per_w)], q_v)
        pltpu.sync_copy(off_hbm.at[pl.ds(base, b_per_w)], off_v)
        liota = lax.iota(jnp.int32, LANES)

        for c in range(b_per_w // CHUNK):
            cbase = c * CHUNK
            pltpu.async_copy(
                table_hbm.at[q_v.at[pl.ds(cbase, CHUNK)]], rows_v, sem
            ).wait()

            def group(k, _, cbase=cbase):
                row_l = k * LANES + liota
                offv = plsc.load_gather(off_v, [cbase + row_l])
                out_row = cbase + row_l
                for j in range(EMBED_DIM):
                    vals = plsc.load_gather(rows_v, [row_l, offv + j])
                    plsc.store_scatter(
                        out_v, [out_row, jnp.full((LANES,), j, jnp.int32)], vals
                    )
                return _

            lax.fori_loop(0, CHUNK // LANES, group, None)
        pltpu.sync_copy(out_v, out_hbm.at[pl.ds(base, b_per_w)])

    return gather_kernel


_gather = _make_gather()


def kernel(in_feat, embed_table):
    idx = in_feat.astype(jnp.int32)
    q = idx >> 2
    off = (idx & 3) << 5
    table128 = embed_table.reshape(VOCAB // PER_ROW, ROW_W)
    return _gather(q, off, table128)


# speedup vs baseline: 1.0416x; 1.0416x over previous
import functools

import jax
import jax.numpy as jnp
from jax import lax
from jax.experimental import pallas as pl
from jax.experimental.pallas import tpu as pltpu
from jax.experimental.pallas import tpu_sc as plsc

VOCAB = 1000000
EMBED_DIM = 32
BATCH = 16384


def _make_gather():
    info = plsc.get_sparse_core_info()
    nc, ns = info.num_cores, info.num_subcores
    nw = nc * ns
    b_per_w = BATCH // nw

    mesh = plsc.VectorSubcoreMesh(core_axis_name="c", subcore_axis_name="s")

    @functools.partial(
        pl.kernel,
        mesh=mesh,
        out_type=jax.ShapeDtypeStruct((BATCH, EMBED_DIM), jnp.float32),
        scratch_types=[
            pltpu.VMEM((b_per_w,), jnp.int32),
            pltpu.VMEM((b_per_w, EMBED_DIM), jnp.float32),
            pltpu.SemaphoreType.DMA,
        ],
        compiler_params=pltpu.CompilerParams(use_tc_tiling_on_sc=False),
    )
    def gather_kernel(idx_hbm, table_hbm, out_hbm, idx_v, rows_v, sem):
        wid = lax.axis_index("s") * nc + lax.axis_index("c")
        base = wid * b_per_w
        pltpu.sync_copy(idx_hbm.at[pl.ds(base, b_per_w)], idx_v)
        pltpu.async_copy(table_hbm.at[idx_v], rows_v, sem).wait()
        pltpu.sync_copy(rows_v, out_hbm.at[pl.ds(base, b_per_w)])

    return gather_kernel


_gather = _make_gather()


def kernel(in_feat, embed_table):
    return _gather(in_feat.astype(jnp.int32), embed_table)
